# Initial kernel scaffold; baseline (speedup 1.0000x reference)
import functools

import jax
import jax.numpy as jnp
from jax import lax
from jax.experimental import pallas as pl
from jax.experimental.pallas import tpu as pltpu
from jax.experimental.pallas import tpu_sc as plsc

N = 10000
D = 128
G = 64
NC = 2
NS = 16
NW = NC * NS
K = 128
NPAD = N + 16
RPS = NPAD // NS
BN = 1000
NB = N // BN

_MESH = functools.partial(
    plsc.VectorSubcoreMesh,
    core_axis_name="c", subcore_axis_name="s", num_cores=NC, num_subcores=NS,
)


def _deg_sc(dstp, ones, zeros):
    nch = dstp.shape[1]

    @functools.partial(
        pl.kernel,
        out_type=jax.ShapeDtypeStruct((NC, NPAD, 16), jnp.float32),
        mesh=_MESH(),
        scratch_types=[
            pltpu.VMEM((nch, K), jnp.int32),
            pltpu.VMEM((K, 16), jnp.float32),
            pltpu.VMEM_SHARED((NPAD, 16), jnp.float32),
        ],
    )
    def body(dst_hbm, ones_hbm, zeros_hbm, out_hbm, dst_v, ones_v, acc_sh):
        c = lax.axis_index("c")
        s = lax.axis_index("s")
        w = c * NS + s
        pltpu.sync_copy(zeros_hbm, acc_sh.at[pl.ds(s * RPS, RPS)])
        pltpu.sync_copy(dst_hbm.at[w], dst_v)
        pltpu.sync_copy(ones_hbm, ones_v)
        plsc.subcore_barrier()

        @pl.loop(0, nch)
        def _(j):
            pltpu.sync_copy(ones_v, acc_sh.at[dst_v.at[j]], add=True)

        plsc.subcore_barrier()
        pltpu.sync_copy(acc_sh.at[pl.ds(s * RPS, RPS)],
                        out_hbm.at[c, pl.ds(s * RPS, RPS)])

    return body(dstp, ones, zeros)


def _agg_sc(h, srcp, dstp, zeros):
    nch = srcp.shape[1]

    @functools.partial(
        pl.kernel,
        out_type=jax.ShapeDtypeStruct((NC, NPAD, D), jnp.float32),
        mesh=_MESH(),
        scratch_types=[
            pltpu.VMEM((nch, K), jnp.int32),
            pltpu.VMEM((nch, K), jnp.int32),
            pltpu.VMEM((K, D), jnp.float32),
            pltpu.VMEM_SHARED((NPAD, D), jnp.float32),
            pltpu.SemaphoreType.DMA,
        ],
    )
    def body(h_hbm, src_hbm, dst_hbm, zeros_hbm, out_hbm,
             src_v, dst_v, buf_v, acc_sh, sem):
        c = lax.axis_index("c")
        s = lax.axis_index("s")
        w = c * NS + s
        pltpu.sync_copy(zeros_hbm, acc_sh.at[pl.ds(s * RPS, RPS)])
        pltpu.sync_copy(src_hbm.at[w], src_v)
        pltpu.sync_copy(dst_hbm.at[w], dst_v)
        plsc.subcore_barrier()

        @pl.loop(0, nch)
        def _(j):
            pltpu.async_copy(h_hbm.at[src_v.at[j]], buf_v, sem).wait()
            pltpu.sync_copy(buf_v, acc_sh.at[dst_v.at[j]], add=True)

        plsc.subcore_barrier()
        pltpu.sync_copy(acc_sh.at[pl.ds(s * RPS, RPS)],
                        out_hbm.at[c, pl.ds(s * RPS, RPS)])

    return body(h, srcp, dstp, zeros)


def _dinv_block(deg_ref):
    d = deg_ref[0, :, 0:1] + deg_ref[1, :, 0:1] + 1.0
    return lax.rsqrt(d)


def _first_tc(x, W1, deg):
    def body(x_ref, w_ref, deg_ref, out_ref):
        dinv = _dinv_block(deg_ref)
        out_ref[...] = dinv * jnp.dot(x_ref[...], w_ref[...],
                                      preferred_element_type=jnp.float32)

    return pl.pallas_call(
        body,
        grid=(NB,),
        in_specs=[
            pl.BlockSpec((BN, D), lambda i: (i, 0)),
            pl.BlockSpec((D, D), lambda i: (0, 0)),
            pl.BlockSpec((NC, BN, 16), lambda i: (0, i, 0)),
        ],
        out_specs=pl.BlockSpec((BN, D), lambda i: (i, 0)),
        out_shape=jax.ShapeDtypeStruct((N, D), jnp.float32),
    )(x, W1, deg)


def _mid_tc(acc, hprev, Wn, b, deg):
    def body(acc_ref, hp_ref, w_ref, b_ref, deg_ref, out_ref):
        dinv = _dinv_block(deg_ref)
        t = dinv * (acc_ref[0] + acc_ref[1] + hp_ref[...]) + b_ref[...]
        u = jnp.maximum(t, 0.0)
        out_ref[...] = dinv * jnp.dot(u, w_ref[...],
                                      preferred_element_type=jnp.float32)

    return pl.pallas_call(
        body,
        grid=(NB,),
        in_specs=[
            pl.BlockSpec((NC, BN, D), lambda i: (0, i, 0)),
            pl.BlockSpec((BN, D), lambda i: (i, 0)),
            pl.BlockSpec((D, D), lambda i: (0, 0)),
            pl.BlockSpec((1, D), lambda i: (0, 0)),
            pl.BlockSpec((NC, BN, 16), lambda i: (0, i, 0)),
        ],
        out_specs=pl.BlockSpec((BN, D), lambda i: (i, 0)),
        out_shape=jax.ShapeDtypeStruct((N, D), jnp.float32),
    )(acc, hprev, Wn, b, deg)


def _pool_tc(acc, hprev, b3, deg, batch_r):
    def body(acc_ref, hp_ref, b_ref, deg_ref, batch_ref, out_ref, sum_s, cnt_s):
        i = pl.program_id(0)
        dinv = _dinv_block(deg_ref)
        h3 = dinv * (acc_ref[0] + acc_ref[1] + hp_ref[...]) + b_ref[...]
        bb = batch_ref[0]
        gi = lax.broadcasted_iota(jnp.int32, (G, BN), 0)
        oh = jnp.where(bb == gi, 1.0, 0.0)
        psum = jnp.dot(oh, h3, preferred_element_type=jnp.float32)
        pcnt = jnp.sum(oh, axis=1, keepdims=True)

        @pl.when(i == 0)
        def _():
            sum_s[...] = jnp.zeros_like(sum_s)
            cnt_s[...] = jnp.zeros_like(cnt_s)

        sum_s[...] = sum_s[...] + psum
        cnt_s[...] = cnt_s[...] + pcnt

        @pl.when(i == NB - 1)
        def _():
            out_ref[...] = sum_s[...] / jnp.maximum(cnt_s[...], 1.0)

    return pl.pallas_call(
        body,
        grid=(NB,),
        in_specs=[
            pl.BlockSpec((NC, BN, D), lambda i: (0, i, 0)),
            pl.BlockSpec((BN, D), lambda i: (i, 0)),
            pl.BlockSpec((1, D), lambda i: (0, 0)),
            pl.BlockSpec((NC, BN, 16), lambda i: (0, i, 0)),
            pl.BlockSpec((1, 1, BN), lambda i: (i, 0, 0)),
        ],
        out_specs=pl.BlockSpec((G, D), lambda i: (0, 0)),
        out_shape=jax.ShapeDtypeStruct((G, D), jnp.float32),
        scratch_shapes=[
            pltpu.VMEM((G, D), jnp.float32),
            pltpu.VMEM((G, 128), jnp.float32),
        ],
    )(acc, hprev, b3, deg, batch_r)


def kernel(x, edge_index, batch, W1, b1, W2, b2, W3, b3):
    E = edge_index.shape[1]
    epw = E // NW
    nch = -(-epw // K)
    pad = nch * K - epw

    src = edge_index[0].reshape(NW, epw)
    dst = edge_index[1].reshape(NW, epw)
    pad_dst = (N + (jnp.arange(pad, dtype=jnp.int32) % 16))[None, :]
    srcp = jnp.concatenate(
        [src, jnp.zeros((NW, pad), jnp.int32)], axis=1).reshape(NW, nch, K)
    dstp = jnp.concatenate(
        [dst, jnp.broadcast_to(pad_dst, (NW, pad))], axis=1).reshape(NW, nch, K)

    ones16 = jnp.ones((K, 16), jnp.float32)
    zeros16 = jnp.zeros((RPS, 16), jnp.float32)
    zerosD = jnp.zeros((RPS, D), jnp.float32)
    b1r = b1.reshape(1, D)
    b2r = b2.reshape(1, D)
    b3r = b3.reshape(1, D)
    batch_r = batch.reshape(NB, 1, BN)

    deg = _deg_sc(dstp, ones16, zeros16)

    h1 = _first_tc(x, W1, deg)
    a1 = _agg_sc(h1, srcp, dstp, zerosD)
    h2 = _mid_tc(a1, h1, W2, b1r, deg)
    a2 = _agg_sc(h2, srcp, dstp, zerosD)
    h3 = _mid_tc(a2, h2, W3, b2r, deg)
    a3 = _agg_sc(h3, srcp, dstp, zerosD)
    return _pool_tc(a3, h3, b3r, deg, batch_r)


# speedup vs baseline: 11.5256x; 11.5256x over previous
import functools

import jax
import jax.numpy as jnp
from jax import lax
from jax.experimental import pallas as pl
from jax.experimental.pallas import tpu as pltpu
from jax.experimental.pallas import tpu_sc as plsc

N = 10000
D = 128
G = 64
NC = 2
NS = 16
NW = NC * NS
K = 128
NPAD = 10112
RPS = NPAD // NS
BN = 1000
NB = N // BN

_MESH = functools.partial(
    plsc.VectorSubcoreMesh,
    core_axis_name="c", subcore_axis_name="s", num_cores=NC, num_subcores=NS,
)


def _deg_sc(dstp, ones, zeros):
    nch = dstp.shape[1]

    @functools.partial(
        pl.kernel,
        out_type=jax.ShapeDtypeStruct((NC, NPAD, D), jnp.float32),
        mesh=_MESH(),
        scratch_types=[
            pltpu.VMEM((nch, K), jnp.int32),
            pltpu.VMEM((K, D), jnp.float32),
            pltpu.VMEM_SHARED((NPAD, D), jnp.float32),
        ],
    )
    def body(dst_hbm, ones_hbm, zeros_hbm, out_hbm, dst_v, ones_v, acc_sh):
        c = lax.axis_index("c")
        s = lax.axis_index("s")
        w = c * NS + s
        pltpu.sync_copy(zeros_hbm, acc_sh.at[pl.ds(s * RPS, RPS)])
        pltpu.sync_copy(dst_hbm.at[w], dst_v)
        pltpu.sync_copy(ones_hbm, ones_v)
        plsc.subcore_barrier()

        @pl.loop(0, nch)
        def _(j):
            pltpu.sync_copy(ones_v, acc_sh.at[dst_v.at[j]], add=True)

        plsc.subcore_barrier()
        pltpu.sync_copy(acc_sh.at[pl.ds(s * RPS, RPS)],
                        out_hbm.at[c, pl.ds(s * RPS, RPS)])

    return body(dstp, ones, zeros)


def _agg_sc(h, srcp, dstp, zeros):
    nch = srcp.shape[1]

    @functools.partial(
        pl.kernel,
        out_type=jax.ShapeDtypeStruct((NC, NPAD, D), jnp.float32),
        mesh=_MESH(),
        scratch_types=[
            pltpu.VMEM((nch, K), jnp.int32),
            pltpu.VMEM((nch, K), jnp.int32),
            pltpu.VMEM((K, D), jnp.float32),
            pltpu.VMEM_SHARED((NPAD, D), jnp.float32),
            pltpu.SemaphoreType.DMA,
        ],
    )
    def body(h_hbm, src_hbm, dst_hbm, zeros_hbm, out_hbm,
             src_v, dst_v, buf_v, acc_sh, sem):
        c = lax.axis_index("c")
        s = lax.axis_index("s")
        w = c * NS + s
        pltpu.sync_copy(zeros_hbm, acc_sh.at[pl.ds(s * RPS, RPS)])
        pltpu.sync_copy(src_hbm.at[w], src_v)
        pltpu.sync_copy(dst_hbm.at[w], dst_v)
        plsc.subcore_barrier()

        @pl.loop(0, nch)
        def _(j):
            pltpu.async_copy(h_hbm.at[src_v.at[j]], buf_v, sem).wait()
            pltpu.sync_copy(buf_v, acc_sh.at[dst_v.at[j]], add=True)

        plsc.subcore_barrier()
        pltpu.sync_copy(acc_sh.at[pl.ds(s * RPS, RPS)],
                        out_hbm.at[c, pl.ds(s * RPS, RPS)])

    return body(h, srcp, dstp, zeros)


def _dinv_block(deg_ref):
    d = deg_ref[0, :, 0:1] + deg_ref[1, :, 0:1] + 1.0
    return lax.rsqrt(d)


def _first_tc(x, W1, deg):
    def body(x_ref, w_ref, deg_ref, out_ref):
        dinv = _dinv_block(deg_ref)
        out_ref[...] = dinv * jnp.dot(x_ref[...], w_ref[...],
                                      preferred_element_type=jnp.float32)

    return pl.pallas_call(
        body,
        grid=(NB,),
        in_specs=[
            pl.BlockSpec((BN, D), lambda i: (i, 0)),
            pl.BlockSpec((D, D), lambda i: (0, 0)),
            pl.BlockSpec((NC, BN, D), lambda i: (0, i, 0)),
        ],
        out_specs=pl.BlockSpec((BN, D), lambda i: (i, 0)),
        out_shape=jax.ShapeDtypeStruct((N, D), jnp.float32),
    )(x, W1, deg)


def _mid_tc(acc, hprev, Wn, b, deg):
    def body(acc_ref, hp_ref, w_ref, b_ref, deg_ref, out_ref):
        dinv = _dinv_block(deg_ref)
        t = dinv * (acc_ref[0] + acc_ref[1] + hp_ref[...]) + b_ref[...]
        u = jnp.maximum(t, 0.0)
        out_ref[...] = dinv * jnp.dot(u, w_ref[...],
                                      preferred_element_type=jnp.float32)

    return pl.pallas_call(
        body,
        grid=(NB,),
        in_specs=[
            pl.BlockSpec((NC, BN, D), lambda i: (0, i, 0)),
            pl.BlockSpec((BN, D), lambda i: (i, 0)),
            pl.BlockSpec((D, D), lambda i: (0, 0)),
            pl.BlockSpec((1, D), lambda i: (0, 0)),
            pl.BlockSpec((NC, BN, D), lambda i: (0, i, 0)),
        ],
        out_specs=pl.BlockSpec((BN, D), lambda i: (i, 0)),
        out_shape=jax.ShapeDtypeStruct((N, D), jnp.float32),
    )(acc, hprev, Wn, b, deg)


def _pool_tc(acc, hprev, b3, deg, batch_r):
    def body(acc_ref, hp_ref, b_ref, deg_ref, batch_ref, out_ref, sum_s, cnt_s):
        i = pl.program_id(0)
        dinv = _dinv_block(deg_ref)
        h3 = dinv * (acc_ref[0] + acc_ref[1] + hp_ref[...]) + b_ref[...]
        bb = batch_ref[0]
        gi = lax.broadcasted_iota(jnp.int32, (G, BN), 0)
        oh = jnp.where(bb == gi, 1.0, 0.0)
        psum = jnp.dot(oh, h3, preferred_element_type=jnp.float32)
        pcnt = jnp.sum(oh, axis=1, keepdims=True)

        @pl.when(i == 0)
        def _():
            sum_s[...] = jnp.zeros_like(sum_s)
            cnt_s[...] = jnp.zeros_like(cnt_s)

        sum_s[...] = sum_s[...] + psum
        cnt_s[...] = cnt_s[...] + pcnt

        @pl.when(i == NB - 1)
        def _():
            out_ref[...] = sum_s[...] / jnp.maximum(cnt_s[...], 1.0)

    return pl.pallas_call(
        body,
        grid=(NB,),
        in_specs=[
            pl.BlockSpec((NC, BN, D), lambda i: (0, i, 0)),
            pl.BlockSpec((BN, D), lambda i: (i, 0)),
            pl.BlockSpec((1, D), lambda i: (0, 0)),
            pl.BlockSpec((NC, BN, D), lambda i: (0, i, 0)),
            pl.BlockSpec((1, 1, BN), lambda i: (i, 0, 0)),
        ],
        out_specs=pl.BlockSpec((G, D), lambda i: (0, 0)),
        out_shape=jax.ShapeDtypeStruct((G, D), jnp.float32),
        scratch_shapes=[
            pltpu.VMEM((G, D), jnp.float32),
            pltpu.VMEM((G, 128), jnp.float32),
        ],
    )(acc, hprev, b3, deg, batch_r)


def kernel(x, edge_index, batch, W1, b1, W2, b2, W3, b3):
    E = edge_index.shape[1]
    epw = E // NW
    nch = -(-epw // K)
    pad = nch * K - epw

    src = edge_index[0].reshape(NW, epw)
    dst = edge_index[1].reshape(NW, epw)
    pad_dst = (N + (jnp.arange(pad, dtype=jnp.int32) % 16))[None, :]
    srcp = jnp.concatenate(
        [src, jnp.zeros((NW, pad), jnp.int32)], axis=1).reshape(NW, nch, K)
    dstp = jnp.concatenate(
        [dst, jnp.broadcast_to(pad_dst, (NW, pad))], axis=1).reshape(NW, nch, K)

    onesD = jnp.ones((K, D), jnp.float32)
    zerosD = jnp.zeros((RPS, D), jnp.float32)
    b1r = b1.reshape(1, D)
    b2r = b2.reshape(1, D)
    b3r = b3.reshape(1, D)
    batch_r = batch.reshape(NB, 1, BN)

    deg = _deg_sc(dstp, onesD, zerosD)

    h1 = _first_tc(x, W1, deg)
    a1 = _agg_sc(h1, srcp, dstp, zerosD)
    h2 = _mid_tc(a1, h1, W2, b1r, deg)
    a2 = _agg_sc(h2, srcp, dstp, zerosD)
    h3 = _mid_tc(a2, h2, W3, b2r, deg)
    a3 = _agg_sc(h3, srcp, dstp, zerosD)
    return _pool_tc(a3, h3, b3r, deg, batch_r)
